# Initial kernel scaffold; baseline (speedup 1.0000x reference)
import jax
import jax.numpy as jnp
from jax.experimental import pallas as pl


def kernel(s, r, o, E_t, R_ht, R_tt, E_b, R_b, img_emb, W_lin, b_lin, gamma, beta):
    raise NotImplementedError("write your pallas kernel here")


# speedup vs baseline: 3.9301x; 3.9301x over previous
import functools

import jax
import jax.numpy as jnp
from jax import lax
from jax.experimental import pallas as pl
from jax.experimental.pallas import tpu as pltpu
from jax.experimental.pallas import tpu_sc as plsc

_PSI = 1.0
_MULT = 20.0
_EPS = 1e-5

_B = 16384
_D = 128
_IMG = 512
_NW = 32
_BPW = _B // _NW
_C = 64
_NCH = _BPW // _C

_BT = 512
_NT = _B // _BT


def _sc_body(s_h, r_h, o_h, et_h, rht_h, rtt_h, eb_h, rb_h, img_h,
             gs_h, go_h, ps_h, po_h, b3_h,
             cs, cr, co, ets, rht, eto, rtt, ebs, rb, ebo, imgs, imgo, sem):
    wid = lax.axis_index("sub") * 2 + lax.axis_index("core")
    base = wid * _BPW

    def chunk(c, carry):
        off = base + c * _C
        pltpu.sync_copy(s_h.at[pl.ds(off, _C)], cs)
        pltpu.sync_copy(r_h.at[pl.ds(off, _C)], cr)
        pltpu.sync_copy(o_h.at[pl.ds(off, _C)], co)
        cps = [
            pltpu.async_copy(et_h.at[cs], ets, sem),
            pltpu.async_copy(rht_h.at[cr], rht, sem),
            pltpu.async_copy(et_h.at[co], eto, sem),
            pltpu.async_copy(rtt_h.at[cr], rtt, sem),
            pltpu.async_copy(eb_h.at[cs], ebs, sem),
            pltpu.async_copy(rb_h.at[cr], rb, sem),
            pltpu.async_copy(eb_h.at[co], ebo, sem),
            pltpu.async_copy(img_h.at[cs], imgs, sem),
            pltpu.async_copy(img_h.at[co], imgo, sem),
        ]
        for cp in cps:
            cp.wait()

        def prow(i, cc):
            for j in range(_D // 16):
                ix = (i, pl.ds(j * 16, 16))
                ets[ix] = ets[ix] * rht[ix]
                eto[ix] = eto[ix] * rtt[ix]
                ebs[ix] = ebs[ix] * rb[ix] * ebo[ix]
            return cc

        lax.fori_loop(0, _C, prow, 0)
        pltpu.sync_copy(ets, ps_h.at[pl.ds(off, _C)])
        pltpu.sync_copy(eto, po_h.at[pl.ds(off, _C)])
        pltpu.sync_copy(ebs, b3_h.at[pl.ds(off, _C)])
        pltpu.sync_copy(imgs, gs_h.at[pl.ds(off, _C)])
        pltpu.sync_copy(imgo, go_h.at[pl.ds(off, _C)])
        return carry

    lax.fori_loop(0, _NCH, chunk, 0)


def _sc_gather(*args):
    fn = pl.kernel(
        _sc_body,
        mesh=plsc.VectorSubcoreMesh(core_axis_name="core", subcore_axis_name="sub"),
        out_type=[
        jax.ShapeDtypeStruct((_B, _IMG), jnp.float32),
        jax.ShapeDtypeStruct((_B, _IMG), jnp.float32),
        jax.ShapeDtypeStruct((_B, _D), jnp.float32),
        jax.ShapeDtypeStruct((_B, _D), jnp.float32),
        jax.ShapeDtypeStruct((_B, _D), jnp.float32),
        ],
        scratch_types=[
            pltpu.VMEM((_C,), jnp.int32),
            pltpu.VMEM((_C,), jnp.int32),
            pltpu.VMEM((_C,), jnp.int32),
            pltpu.VMEM((_C, _D), jnp.float32),
            pltpu.VMEM((_C, _D), jnp.float32),
            pltpu.VMEM((_C, _D), jnp.float32),
            pltpu.VMEM((_C, _D), jnp.float32),
            pltpu.VMEM((_C, _D), jnp.float32),
            pltpu.VMEM((_C, _D), jnp.float32),
            pltpu.VMEM((_C, _D), jnp.float32),
            pltpu.VMEM((_C, _IMG), jnp.float32),
            pltpu.VMEM((_C, _IMG), jnp.float32),
            pltpu.SemaphoreType.DMA,
        ],
    )
    return fn(*args)


def _tc_body(gs, go, ps, po, b3, w, bl, gm, bt, out, tmp_s, tmp_o, acc):
    p = pl.program_id(0)
    t = pl.program_id(1)

    @pl.when(p == 0)
    def _pass1():
        ts = lax.dot_general(gs[...], w[...], (((1,), (1,)), ((), ())),
                             preferred_element_type=jnp.float32,
                             precision=lax.Precision.HIGHEST) + bl[...]
        to = lax.dot_general(go[...], w[...], (((1,), (1,)), ((), ())),
                             preferred_element_type=jnp.float32,
                             precision=lax.Precision.HIGHEST) + bl[...]
        tmp_s[pl.ds(t * _BT, _BT), :] = ts
        tmp_o[pl.ds(t * _BT, _BT), :] = to

        @pl.when(t == 0)
        def _init():
            acc[...] = jnp.zeros_like(acc)

        delta = jnp.concatenate([
            jnp.sum(ts, axis=0, keepdims=True),
            jnp.sum(ts * ts, axis=0, keepdims=True),
            jnp.sum(to, axis=0, keepdims=True),
            jnp.sum(to * to, axis=0, keepdims=True),
        ], axis=0)
        acc[...] = acc[...] + delta

    @pl.when(p == 1)
    def _pass2():
        a = acc[...]
        inv_b = 1.0 / _B
        mean_s = a[0:1] * inv_b
        var_s = a[1:2] * inv_b - mean_s * mean_s
        mean_o = a[2:3] * inv_b
        var_o = a[3:4] * inv_b - mean_o * mean_o
        scale_s = gm[...] * lax.rsqrt(var_s + _EPS)
        shift_s = bt[...] - mean_s * scale_s
        scale_o = gm[...] * lax.rsqrt(var_o + _EPS)
        shift_o = bt[...] - mean_o * scale_o

        ts = tmp_s[pl.ds(t * _BT, _BT), :] * scale_s + shift_s
        to = tmp_o[pl.ds(t * _BT, _BT), :] * scale_o + shift_o
        psv = ps[...]
        pov = po[...]
        base = jnp.sum(b3[...], axis=1)
        ht = jnp.sum(psv, axis=1)
        tt = jnp.sum(pov, axis=1)
        ih = jnp.sum(ts * psv, axis=1)
        it = jnp.sum(to * pov, axis=1)
        ii = jnp.sum(ts * to, axis=1)
        sig = lambda x: jax.nn.sigmoid(_PSI * x)
        out[...] = _MULT * (sig(base) * sig(ht) * sig(tt)
                            + 0.005 * (ih + it + ii))


def _tc_finish(gs, go, ps, po, b3, w, bl, gm, bt):
    return pl.pallas_call(
        _tc_body,
        grid=(2, _NT),
        in_specs=[
            pl.BlockSpec((_BT, _IMG), lambda p, t: ((1 - p) * t, 0)),
            pl.BlockSpec((_BT, _IMG), lambda p, t: ((1 - p) * t, 0)),
            pl.BlockSpec((_BT, _D), lambda p, t: (p * t, 0)),
            pl.BlockSpec((_BT, _D), lambda p, t: (p * t, 0)),
            pl.BlockSpec((_BT, _D), lambda p, t: (p * t, 0)),
            pl.BlockSpec((_D, _IMG), lambda p, t: (0, 0)),
            pl.BlockSpec((1, _D), lambda p, t: (0, 0)),
            pl.BlockSpec((1, _D), lambda p, t: (0, 0)),
            pl.BlockSpec((1, _D), lambda p, t: (0, 0)),
        ],
        out_specs=pl.BlockSpec((_BT,), lambda p, t: (p * t,)),
        out_shape=jax.ShapeDtypeStruct((_B,), jnp.float32),
        scratch_shapes=[
            pltpu.VMEM((_B, _D), jnp.float32),
            pltpu.VMEM((_B, _D), jnp.float32),
            pltpu.VMEM((4, _D), jnp.float32),
        ],
        compiler_params=pltpu.CompilerParams(
            dimension_semantics=("arbitrary", "arbitrary")),
    )(gs, go, ps, po, b3, w, bl, gm, bt)


def kernel(s, r, o, E_t, R_ht, R_tt, E_b, R_b, img_emb, W_lin, b_lin, gamma, beta):
    s1 = s.reshape(-1).astype(jnp.int32)
    r1 = r.reshape(-1).astype(jnp.int32)
    o1 = o.reshape(-1).astype(jnp.int32)
    gs, go, ps, po, b3 = _sc_gather(
        s1, r1, o1, E_t, R_ht, R_tt, E_b, R_b, img_emb)
    res = _tc_finish(gs, go, ps, po, b3, W_lin,
                     b_lin.reshape(1, _D), gamma.reshape(1, _D),
                     beta.reshape(1, _D))
    return res.reshape(_B, 1)


# speedup vs baseline: 4.5338x; 1.1536x over previous
import functools

import jax
import jax.numpy as jnp
from jax import lax
from jax.experimental import pallas as pl
from jax.experimental.pallas import tpu as pltpu
from jax.experimental.pallas import tpu_sc as plsc

_PSI = 1.0
_MULT = 20.0
_EPS = 1e-5

_B = 16384
_D = 128
_IMG = 512
_NW = 32
_BPW = _B // _NW
_C = 64
_NCH = _BPW // _C

_BT = 512
_NT = _B // _BT


def _sc_body(s_h, r_h, o_h, et_h, rht_h, rtt_h, eb_h, rb_h, img_h,
             gs_h, go_h, ps_h, po_h, b3_h,
             cs, cr, co, ets, rht, eto, rtt, ebs, rb, ebo, imgs, imgo, sem):
    wid = lax.axis_index("sub") * 2 + lax.axis_index("core")
    base = wid * _BPW

    def chunk(c, carry):
        off = base + c * _C
        pltpu.sync_copy(s_h.at[pl.ds(off, _C)], cs)
        pltpu.sync_copy(r_h.at[pl.ds(off, _C)], cr)
        pltpu.sync_copy(o_h.at[pl.ds(off, _C)], co)
        cps = [
            pltpu.async_copy(et_h.at[cs], ets, sem),
            pltpu.async_copy(rht_h.at[cr], rht, sem),
            pltpu.async_copy(et_h.at[co], eto, sem),
            pltpu.async_copy(rtt_h.at[cr], rtt, sem),
            pltpu.async_copy(eb_h.at[cs], ebs, sem),
            pltpu.async_copy(rb_h.at[cr], rb, sem),
            pltpu.async_copy(eb_h.at[co], ebo, sem),
            pltpu.async_copy(img_h.at[cs], imgs, sem),
            pltpu.async_copy(img_h.at[co], imgo, sem),
        ]
        for cp in cps:
            cp.wait()

        def prow(i, cc):
            for j in range(_D // 16):
                ix = (i, pl.ds(j * 16, 16))
                ets[ix] = ets[ix] * rht[ix]
                eto[ix] = eto[ix] * rtt[ix]
                ebs[ix] = ebs[ix] * rb[ix] * ebo[ix]
            return cc

        lax.fori_loop(0, _C, prow, 0)
        pltpu.sync_copy(ets, ps_h.at[pl.ds(off, _C)])
        pltpu.sync_copy(eto, po_h.at[pl.ds(off, _C)])
        pltpu.sync_copy(ebs, b3_h.at[pl.ds(off, _C)])
        pltpu.sync_copy(imgs, gs_h.at[pl.ds(off, _C)])
        pltpu.sync_copy(imgo, go_h.at[pl.ds(off, _C)])
        return carry

    lax.fori_loop(0, _NCH, chunk, 0)


def _sc_gather(*args):
    fn = pl.kernel(
        _sc_body,
        mesh=plsc.VectorSubcoreMesh(core_axis_name="core", subcore_axis_name="sub"),
        out_type=[
        jax.ShapeDtypeStruct((_B, _IMG), jnp.float32),
        jax.ShapeDtypeStruct((_B, _IMG), jnp.float32),
        jax.ShapeDtypeStruct((_B, _D), jnp.float32),
        jax.ShapeDtypeStruct((_B, _D), jnp.float32),
        jax.ShapeDtypeStruct((_B, _D), jnp.float32),
        ],
        scratch_types=[
            pltpu.VMEM((_C,), jnp.int32),
            pltpu.VMEM((_C,), jnp.int32),
            pltpu.VMEM((_C,), jnp.int32),
            pltpu.VMEM((_C, _D), jnp.float32),
            pltpu.VMEM((_C, _D), jnp.float32),
            pltpu.VMEM((_C, _D), jnp.float32),
            pltpu.VMEM((_C, _D), jnp.float32),
            pltpu.VMEM((_C, _D), jnp.float32),
            pltpu.VMEM((_C, _D), jnp.float32),
            pltpu.VMEM((_C, _D), jnp.float32),
            pltpu.VMEM((_C, _IMG), jnp.float32),
            pltpu.VMEM((_C, _IMG), jnp.float32),
            pltpu.SemaphoreType.DMA,
        ],
    )
    return fn(*args)


def _tc_body(gs, go, ps, po, b3, w, bl, gm, bt, out, tmp_s, tmp_o, acc):
    p = pl.program_id(0)
    t = pl.program_id(1)

    @pl.when(p == 0)
    def _pass1():
        wb = w[...].astype(jnp.bfloat16)
        ts = lax.dot_general(gs[...].astype(jnp.bfloat16), wb,
                             (((1,), (1,)), ((), ())),
                             preferred_element_type=jnp.float32) + bl[...]
        to = lax.dot_general(go[...].astype(jnp.bfloat16), wb,
                             (((1,), (1,)), ((), ())),
                             preferred_element_type=jnp.float32) + bl[...]
        tmp_s[pl.ds(t * _BT, _BT), :] = ts
        tmp_o[pl.ds(t * _BT, _BT), :] = to

        @pl.when(t == 0)
        def _init():
            acc[...] = jnp.zeros_like(acc)

        delta = jnp.concatenate([
            jnp.sum(ts, axis=0, keepdims=True),
            jnp.sum(ts * ts, axis=0, keepdims=True),
            jnp.sum(to, axis=0, keepdims=True),
            jnp.sum(to * to, axis=0, keepdims=True),
        ], axis=0)
        acc[...] = acc[...] + delta

    @pl.when(p == 1)
    def _pass2():
        a = acc[...]
        inv_b = 1.0 / _B
        mean_s = a[0:1] * inv_b
        var_s = a[1:2] * inv_b - mean_s * mean_s
        mean_o = a[2:3] * inv_b
        var_o = a[3:4] * inv_b - mean_o * mean_o
        scale_s = gm[...] * lax.rsqrt(var_s + _EPS)
        shift_s = bt[...] - mean_s * scale_s
        scale_o = gm[...] * lax.rsqrt(var_o + _EPS)
        shift_o = bt[...] - mean_o * scale_o

        ts = tmp_s[pl.ds(t * _BT, _BT), :] * scale_s + shift_s
        to = tmp_o[pl.ds(t * _BT, _BT), :] * scale_o + shift_o
        psv = ps[...]
        pov = po[...]
        base = jnp.sum(b3[...], axis=1)
        ht = jnp.sum(psv, axis=1)
        tt = jnp.sum(pov, axis=1)
        ih = jnp.sum(ts * psv, axis=1)
        it = jnp.sum(to * pov, axis=1)
        ii = jnp.sum(ts * to, axis=1)
        sig = lambda x: jax.nn.sigmoid(_PSI * x)
        out[...] = _MULT * (sig(base) * sig(ht) * sig(tt)
                            + 0.005 * (ih + it + ii))


def _tc_finish(gs, go, ps, po, b3, w, bl, gm, bt):
    return pl.pallas_call(
        _tc_body,
        grid=(2, _NT),
        in_specs=[
            pl.BlockSpec((_BT, _IMG), lambda p, t: ((1 - p) * t, 0)),
            pl.BlockSpec((_BT, _IMG), lambda p, t: ((1 - p) * t, 0)),
            pl.BlockSpec((_BT, _D), lambda p, t: (p * t, 0)),
            pl.BlockSpec((_BT, _D), lambda p, t: (p * t, 0)),
            pl.BlockSpec((_BT, _D), lambda p, t: (p * t, 0)),
            pl.BlockSpec((_D, _IMG), lambda p, t: (0, 0)),
            pl.BlockSpec((1, _D), lambda p, t: (0, 0)),
            pl.BlockSpec((1, _D), lambda p, t: (0, 0)),
            pl.BlockSpec((1, _D), lambda p, t: (0, 0)),
        ],
        out_specs=pl.BlockSpec((_BT,), lambda p, t: (p * t,)),
        out_shape=jax.ShapeDtypeStruct((_B,), jnp.float32),
        scratch_shapes=[
            pltpu.VMEM((_B, _D), jnp.float32),
            pltpu.VMEM((_B, _D), jnp.float32),
            pltpu.VMEM((4, _D), jnp.float32),
        ],
        compiler_params=pltpu.CompilerParams(
            dimension_semantics=("arbitrary", "arbitrary")),
    )(gs, go, ps, po, b3, w, bl, gm, bt)


def kernel(s, r, o, E_t, R_ht, R_tt, E_b, R_b, img_emb, W_lin, b_lin, gamma, beta):
    s1 = s.reshape(-1).astype(jnp.int32)
    r1 = r.reshape(-1).astype(jnp.int32)
    o1 = o.reshape(-1).astype(jnp.int32)
    gs, go, ps, po, b3 = _sc_gather(
        s1, r1, o1, E_t, R_ht, R_tt, E_b, R_b, img_emb)
    res = _tc_finish(gs, go, ps, po, b3, W_lin,
                     b_lin.reshape(1, _D), gamma.reshape(1, _D),
                     beta.reshape(1, _D))
    return res.reshape(_B, 1)
